# baseline (speedup 1.0000x reference)
import jax
import jax.numpy as jnp
from jax import lax
from jax.experimental import pallas as pl
from jax.experimental.pallas import tpu as pltpu
from jax.experimental.pallas import tpu_sc as plsc

NR, NT, NZ = 128, 256, 32
R_MIN, R_MAX = 0.0, 50.0
Z_MIN, Z_MAX = -10.0, 10.0
C = 16
V = NR * NT * NZ

PTS_PAD = 409600
NPT = PTS_PAD // 16
SEG = 3200
NSEG = NPT // SEG
SVR = SEG // 16
NCH = 16
CH = V // NCH
ROUNDS = NCH // 2
SLICE = CH // 16
TRASH = CH
BIG = 1 << 30

RING = 8192
FLUSH = 1024


def _binning_body(x_ref, y_ref, z_ref, lin_ref):
    x = x_ref[...]
    y = y_ref[...]
    z = z_ref[...]
    r = jnp.sqrt(x * x + y * y)
    theta = jnp.arctan2(y, x)
    theta = (theta + 2.0 * jnp.pi) % (2.0 * jnp.pi)
    ir = jnp.floor((r - R_MIN) / (R_MAX - R_MIN) * NR).astype(jnp.int32)
    it = jnp.floor(theta / (2.0 * jnp.pi) * NT).astype(jnp.int32) % NT
    iz = jnp.floor((z - Z_MIN) / (Z_MAX - Z_MIN) * NZ).astype(jnp.int32)
    valid = (ir >= 0) & (ir < NR) & (iz >= 0) & (iz < NZ)
    irc = jnp.clip(ir, 0, NR - 1)
    itc = jnp.clip(it, 0, NT - 1)
    izc = jnp.clip(iz, 0, NZ - 1)
    lin = (irc * NT + itc) * NZ + izc
    lin_ref[...] = jnp.where(valid, lin, BIG)


def _div_body(gs_ref, occ_ref, out_ref):
    occ = occ_ref[0]
    rcp = jnp.where(occ > 0, 1.0 / (occ + 1e-8), 0.0)
    for z in range(NZ):
        gz = gs_ref[0, :, z * C:(z + 1) * C]
        out_ref[0, z] = (gz * rcp[:, z:z + 1]).T


def _sc_body(lin_hbm, feats_hbm, gsum_hbm, occ_hbm,
             seg_v, cvox, cpid, idx_s, pid_s, grows, ones, zbuf, zocc,
             sfeat, socc, sem):
    cid = lax.axis_index("c")
    sid = lax.axis_index("s")
    pt_base = sid * NPT
    row0 = sid * SLICE
    iota = jnp.arange(16, dtype=jnp.int32)
    zf = jnp.zeros((16,), jnp.float32)
    chv = jnp.full((16,), CH, jnp.uint32)

    def fill_body(i, _):
        zbuf[i, :] = zf
        zocc[pl.ds(i * 16, 16)] = zf
        return 0
    lax.fori_loop(0, 64, fill_body, 0)
    for k in range(FLUSH // 16):
        ones[pl.ds(k * 16, 16)] = jnp.ones((16,), jnp.float32)

    def zero_slice():
        for j in range(SLICE // 64):
            pltpu.sync_copy(zbuf, sfeat.at[pl.ds(row0 + j * 64, 64)])
        for j in range(SLICE // 1024):
            pltpu.sync_copy(zocc, socc.at[pl.ds(row0 + j * 1024, 1024)])

    zero_slice()

    def flush_block(fl):
        o = fl & (RING - 1)
        for k in range(FLUSH // 16):
            idx_s[pl.ds(k * 16, 16)] = cvox[pl.ds(o + k * 16, 16)]
            pid_s[pl.ds(k * 16, 16)] = cpid[pl.ds(o + k * 16, 16)]
        pltpu.async_copy(feats_hbm.at[pid_s], grows, sem).wait()
        pltpu.sync_copy(grows, sfeat.at[idx_s], add=True)
        pltpu.sync_copy(ones, socc.at[idx_s], add=True)

    def round_body(rnd, _):
        plsc.subcore_barrier()
        base = (rnd * 2 + cid) * CH
        base_v = jnp.full((16,), base, jnp.int32)
        ringm = jnp.full((16,), RING - 1, jnp.int32)

        def seg_body(g, carry):
            cnt0, fl0 = carry
            seg_pt = pt_base + g * SEG
            pltpu.sync_copy(lin_hbm.at[pl.ds(seg_pt, SEG)], seg_v)

            def scan_body(i, cnt):
                l = seg_v[pl.ds(i * 16, 16)]
                d = l - base_v
                m = d.astype(jnp.uint32) < chv
                inc = m.astype(jnp.int32)
                pos = (jnp.full((16,), cnt, jnp.int32)
                       + plsc.cumsum(inc) - inc) & ringm
                plsc.store_scatter(cvox, [pos], d, mask=m)
                plsc.store_scatter(cpid, [pos],
                                   iota + jnp.full((16,), seg_pt + i * 16,
                                                   jnp.int32), mask=m)
                return cnt + jnp.sum(inc)
            cnt = lax.fori_loop(0, SVR, scan_body, cnt0)

            def fl_body(j, fl):
                flush_block(fl)
                return fl + FLUSH
            nfl = (cnt - fl0) // FLUSH
            fl = lax.fori_loop(0, nfl, fl_body, fl0)
            return (cnt, fl)

        cnt, fl = lax.fori_loop(0, NSEG, seg_body,
                                (jnp.int32(0), jnp.int32(0)))

        @pl.when(cnt > fl)
        def _():
            trash_v = jnp.full((16,), TRASH, jnp.int32)
            zero_i = jnp.zeros((16,), jnp.int32)
            for k in range(FLUSH // 16):
                posk = (jnp.full((16,), cnt + k * 16, jnp.int32)
                        + iota) & ringm
                plsc.store_scatter(cvox, [posk], trash_v)
                plsc.store_scatter(cpid, [posk], zero_i)
            flush_block(fl)

        plsc.subcore_barrier()

        pltpu.sync_copy(sfeat.at[pl.ds(row0, SLICE)],
                        gsum_hbm.at[pl.ds(base + row0, SLICE)])
        pltpu.sync_copy(socc.at[pl.ds(row0, SLICE)],
                        occ_hbm.at[pl.ds(base + row0, SLICE)])
        zero_slice()
        return 0

    lax.fori_loop(0, ROUNDS, round_body, 0)


_sc_voxelize = pl.kernel(
    _sc_body,
    out_type=[
        jax.ShapeDtypeStruct((V, C), jnp.float32),
        jax.ShapeDtypeStruct((V,), jnp.float32),
    ],
    mesh=plsc.VectorSubcoreMesh(core_axis_name="c", subcore_axis_name="s"),
    compiler_params=pltpu.CompilerParams(needs_layout_passes=False,
                                         use_tc_tiling_on_sc=False),
    scratch_types=[
        pltpu.VMEM((SEG,), jnp.int32),
        pltpu.VMEM((RING,), jnp.int32),
        pltpu.VMEM((RING,), jnp.int32),
        pltpu.VMEM((FLUSH,), jnp.int32),
        pltpu.VMEM((FLUSH,), jnp.int32),
        pltpu.VMEM((FLUSH, C), jnp.float32),
        pltpu.VMEM((FLUSH,), jnp.float32),
        pltpu.VMEM((64, C), jnp.float32),
        pltpu.VMEM((1024,), jnp.float32),
        pltpu.VMEM_SHARED((CH + 16, C), jnp.float32),
        pltpu.VMEM_SHARED((CH + 16,), jnp.float32),
        pltpu.SemaphoreType.DMA,
    ],
)


def kernel(points, features):
    pad = PTS_PAD - points.shape[0]
    x = jnp.pad(points[:, 0], (0, pad), constant_values=1e9)
    y = jnp.pad(points[:, 1], (0, pad), constant_values=1e9)
    z = jnp.pad(points[:, 2], (0, pad), constant_values=1e9)

    BLK = 4096
    lin = pl.pallas_call(
        _binning_body,
        out_shape=jax.ShapeDtypeStruct((PTS_PAD,), jnp.int32),
        grid=(PTS_PAD // BLK,),
        in_specs=[pl.BlockSpec((BLK,), lambda i: (i,))] * 3,
        out_specs=pl.BlockSpec((BLK,), lambda i: (i,)),
    )(x, y, z)

    gsum, occ = _sc_voxelize(lin, features)

    gs3 = gsum.reshape(NR, NT, NZ * C)
    occ3 = occ.reshape(NR, NT, NZ)
    grid_zct = pl.pallas_call(
        _div_body,
        out_shape=jax.ShapeDtypeStruct((NR, NZ, C, NT), jnp.float32),
        grid=(NR,),
        in_specs=[
            pl.BlockSpec((1, NT, NZ * C), lambda i: (i, 0, 0)),
            pl.BlockSpec((1, NT, NZ), lambda i: (i, 0, 0)),
        ],
        out_specs=pl.BlockSpec((1, NZ, C, NT), lambda i: (i, 0, 0, 0)),
    )(gs3, occ3)
    return grid_zct.transpose(0, 3, 1, 2)


# speedup vs baseline: 1.2824x; 1.2824x over previous
import jax
import jax.numpy as jnp
from jax import lax
from jax.experimental import pallas as pl
from jax.experimental.pallas import tpu as pltpu
from jax.experimental.pallas import tpu_sc as plsc

NR, NT, NZ = 128, 256, 32
R_MIN, R_MAX = 0.0, 50.0
Z_MIN, Z_MAX = -10.0, 10.0
C = 16
V = NR * NT * NZ

PTS_PAD = 409600
NPT = PTS_PAD // 16
SEG = 3200
NSEG = NPT // SEG
SVR = SEG // 16
NCH = 16
CH = V // NCH
ROUNDS = NCH // 2
SLICE = CH // 16
TRASH = CH
BIG = 1 << 30

RING = 8192
FLUSH = 512


def _binning_body(x_ref, y_ref, z_ref, lin_ref):
    x = x_ref[...]
    y = y_ref[...]
    z = z_ref[...]
    r = jnp.sqrt(x * x + y * y)
    theta = jnp.arctan2(y, x)
    theta = (theta + 2.0 * jnp.pi) % (2.0 * jnp.pi)
    ir = jnp.floor((r - R_MIN) / (R_MAX - R_MIN) * NR).astype(jnp.int32)
    it = jnp.floor(theta / (2.0 * jnp.pi) * NT).astype(jnp.int32) % NT
    iz = jnp.floor((z - Z_MIN) / (Z_MAX - Z_MIN) * NZ).astype(jnp.int32)
    valid = (ir >= 0) & (ir < NR) & (iz >= 0) & (iz < NZ)
    irc = jnp.clip(ir, 0, NR - 1)
    itc = jnp.clip(it, 0, NT - 1)
    izc = jnp.clip(iz, 0, NZ - 1)
    lin = (irc * NT + itc) * NZ + izc
    lin_ref[...] = jnp.where(valid, lin, BIG)


def _div_body(gs_ref, occ_ref, out_ref):
    occ = occ_ref[0]
    rcp = jnp.where(occ > 0, 1.0 / (occ + 1e-8), 0.0)
    for z in range(NZ):
        gz = gs_ref[0, :, z * C:(z + 1) * C]
        out_ref[0, z] = (gz * rcp[:, z:z + 1]).T


def _sc_body(lin_hbm, feats_hbm, gsum_hbm, occ_hbm,
             seg_v, cvox, cpid, idx0, pid0, idx1, pid1, grows0, grows1,
             ones, zbuf, zocc, sfeat, socc, gsem, lsem):
    cid = lax.axis_index("c")
    sid = lax.axis_index("s")
    pt_base = sid * NPT
    row0 = sid * SLICE
    iota = jnp.arange(16, dtype=jnp.int32)
    zf = jnp.zeros((16,), jnp.float32)
    chv = jnp.full((16,), CH, jnp.uint32)

    def fill_body(i, _):
        zbuf[i, :] = zf
        zocc[pl.ds(i * 16, 16)] = zf
        return 0
    lax.fori_loop(0, 64, fill_body, 0)
    for k in range(FLUSH // 16):
        ones[pl.ds(k * 16, 16)] = jnp.ones((16,), jnp.float32)

    def zero_slice():
        for j in range(SLICE // 64):
            pltpu.sync_copy(zbuf, sfeat.at[pl.ds(row0 + j * 64, 64)])
        for j in range(SLICE // 1024):
            pltpu.sync_copy(zocc, socc.at[pl.ds(row0 + j * 1024, 1024)])

    zero_slice()

    def build_block(b, idx_s, pid_s):
        o = (b * FLUSH) & (RING - 1)
        for k in range(FLUSH // 16):
            idx_s[pl.ds(k * 16, 16)] = cvox[pl.ds(o + k * 16, 16)]
            pid_s[pl.ds(k * 16, 16)] = cpid[pl.ds(o + k * 16, 16)]

    def fire_block(b):
        def go(idx_s, pid_s, grows):
            build_block(b, idx_s, pid_s)
            pltpu.async_copy(feats_hbm.at[pid_s], grows, gsem)

        @pl.when(b % 2 == 0)
        def _():
            go(idx0, pid0, grows0)

        @pl.when(b % 2 == 1)
        def _():
            go(idx1, pid1, grows1)

    def drain_block(b):
        def go(idx_s, pid_s, grows):
            pltpu.make_async_copy(feats_hbm.at[pid_s], grows, gsem).wait()
            pltpu.sync_copy(grows, sfeat.at[idx_s], add=True)
            pltpu.sync_copy(ones, socc.at[idx_s], add=True)

        @pl.when(b % 2 == 0)
        def _():
            go(idx0, pid0, grows0)

        @pl.when(b % 2 == 1)
        def _():
            go(idx1, pid1, grows1)

    def lin_slot(g):
        return g % 2

    def fire_lin(g):
        @pl.when(g < NSEG)
        def _():
            pltpu.async_copy(
                lin_hbm.at[pl.ds(pt_base + g * SEG, SEG)],
                seg_v.at[lin_slot(g)], lsem)

    def wait_lin(g):
        pltpu.make_async_copy(
            lin_hbm.at[pl.ds(pt_base + g * SEG, SEG)],
            seg_v.at[lin_slot(g)], lsem).wait()

    def round_body(rnd, _):
        plsc.subcore_barrier()
        base = (rnd * 2 + cid) * CH
        base_v = jnp.full((16,), base, jnp.int32)
        ringm = jnp.full((16,), RING - 1, jnp.int32)

        fire_lin(0)

        def seg_body(g, carry):
            cnt0, fired0, drained0 = carry
            wait_lin(g)
            fire_lin(g + 1)
            slot = lin_slot(g)
            seg_pt = pt_base + g * SEG

            def scan_body(i, cnt):
                l = seg_v[slot, pl.ds(i * 16, 16)]
                d = l - base_v
                m = d.astype(jnp.uint32) < chv
                inc = m.astype(jnp.int32)
                pos = (jnp.full((16,), cnt, jnp.int32)
                       + plsc.cumsum(inc) - inc) & ringm
                plsc.store_scatter(cvox, [pos], d, mask=m)
                plsc.store_scatter(cpid, [pos],
                                   iota + jnp.full((16,), seg_pt + i * 16,
                                                   jnp.int32), mask=m)
                return cnt + jnp.sum(inc)
            cnt = lax.fori_loop(0, SVR, scan_body, cnt0)

            def fl_body(j, carry):
                fired, drained = carry
                fire_block(fired)
                @pl.when(fired > drained)
                def _():
                    drain_block(drained)
                return (fired + 1,
                        jnp.where(fired > drained, drained + 1, drained))
            nfl = cnt // FLUSH - fired0
            fired, drained = lax.fori_loop(0, nfl, fl_body,
                                           (fired0, drained0))
            return (cnt, fired, drained)

        cnt, fired, drained = lax.fori_loop(
            0, NSEG, seg_body, (jnp.int32(0), jnp.int32(0), jnp.int32(0)))

        @pl.when(fired > drained)
        def _():
            drain_block(drained)

        @pl.when(cnt > fired * FLUSH)
        def _():
            trash_v = jnp.full((16,), TRASH, jnp.int32)
            zero_i = jnp.zeros((16,), jnp.int32)
            for k in range(FLUSH // 16):
                posk = (jnp.full((16,), cnt + k * 16, jnp.int32)
                        + iota) & ringm
                plsc.store_scatter(cvox, [posk], trash_v)
                plsc.store_scatter(cpid, [posk], zero_i)
            fire_block(fired)
            drain_block(fired)

        plsc.subcore_barrier()

        pltpu.sync_copy(sfeat.at[pl.ds(row0, SLICE)],
                        gsum_hbm.at[pl.ds(base + row0, SLICE)])
        pltpu.sync_copy(socc.at[pl.ds(row0, SLICE)],
                        occ_hbm.at[pl.ds(base + row0, SLICE)])
        zero_slice()
        return 0

    lax.fori_loop(0, ROUNDS, round_body, 0)


_sc_voxelize = pl.kernel(
    _sc_body,
    out_type=[
        jax.ShapeDtypeStruct((V, C), jnp.float32),
        jax.ShapeDtypeStruct((V,), jnp.float32),
    ],
    mesh=plsc.VectorSubcoreMesh(core_axis_name="c", subcore_axis_name="s"),
    compiler_params=pltpu.CompilerParams(needs_layout_passes=False,
                                         use_tc_tiling_on_sc=False),
    scratch_types=[
        pltpu.VMEM((2, SEG), jnp.int32),
        pltpu.VMEM((RING,), jnp.int32),
        pltpu.VMEM((RING,), jnp.int32),
        pltpu.VMEM((FLUSH,), jnp.int32),
        pltpu.VMEM((FLUSH,), jnp.int32),
        pltpu.VMEM((FLUSH,), jnp.int32),
        pltpu.VMEM((FLUSH,), jnp.int32),
        pltpu.VMEM((FLUSH, C), jnp.float32),
        pltpu.VMEM((FLUSH, C), jnp.float32),
        pltpu.VMEM((FLUSH,), jnp.float32),
        pltpu.VMEM((64, C), jnp.float32),
        pltpu.VMEM((1024,), jnp.float32),
        pltpu.VMEM_SHARED((CH + 16, C), jnp.float32),
        pltpu.VMEM_SHARED((CH + 16,), jnp.float32),
        pltpu.SemaphoreType.DMA,
        pltpu.SemaphoreType.DMA,
    ],
)


def kernel(points, features):
    pad = PTS_PAD - points.shape[0]
    x = jnp.pad(points[:, 0], (0, pad), constant_values=1e9)
    y = jnp.pad(points[:, 1], (0, pad), constant_values=1e9)
    z = jnp.pad(points[:, 2], (0, pad), constant_values=1e9)

    BLK = 4096
    lin = pl.pallas_call(
        _binning_body,
        out_shape=jax.ShapeDtypeStruct((PTS_PAD,), jnp.int32),
        grid=(PTS_PAD // BLK,),
        in_specs=[pl.BlockSpec((BLK,), lambda i: (i,))] * 3,
        out_specs=pl.BlockSpec((BLK,), lambda i: (i,)),
    )(x, y, z)

    gsum, occ = _sc_voxelize(lin, features)

    gs3 = gsum.reshape(NR, NT, NZ * C)
    occ3 = occ.reshape(NR, NT, NZ)
    grid_zct = pl.pallas_call(
        _div_body,
        out_shape=jax.ShapeDtypeStruct((NR, NZ, C, NT), jnp.float32),
        grid=(NR,),
        in_specs=[
            pl.BlockSpec((1, NT, NZ * C), lambda i: (i, 0, 0)),
            pl.BlockSpec((1, NT, NZ), lambda i: (i, 0, 0)),
        ],
        out_specs=pl.BlockSpec((1, NZ, C, NT), lambda i: (i, 0, 0, 0)),
    )(gs3, occ3)
    return grid_zct.transpose(0, 3, 1, 2)


# speedup vs baseline: 1.5009x; 1.1704x over previous
import jax
import jax.numpy as jnp
from jax import lax
from jax.experimental import pallas as pl
from jax.experimental.pallas import tpu as pltpu
from jax.experimental.pallas import tpu_sc as plsc

NR, NT, NZ = 128, 256, 32
R_MIN, R_MAX = 0.0, 50.0
Z_MIN, Z_MAX = -10.0, 10.0
C = 16
V = NR * NT * NZ

PTS_PAD = 409600
NPT = PTS_PAD // 16
SEG = 3200
NSEG = NPT // SEG
SVR = SEG // 16
NCH = 16
CH = V // NCH
ROUNDS = NCH // 2
SLICE = CH // 16
TRASH = CH
BIG = 1 << 30

RING = 8192
FLUSH = 512


def _binning_body(x_ref, y_ref, z_ref, lin_ref):
    x = x_ref[...]
    y = y_ref[...]
    z = z_ref[...]
    r = jnp.sqrt(x * x + y * y)
    theta = jnp.arctan2(y, x)
    theta = (theta + 2.0 * jnp.pi) % (2.0 * jnp.pi)
    ir = jnp.floor((r - R_MIN) / (R_MAX - R_MIN) * NR).astype(jnp.int32)
    it = jnp.floor(theta / (2.0 * jnp.pi) * NT).astype(jnp.int32) % NT
    iz = jnp.floor((z - Z_MIN) / (Z_MAX - Z_MIN) * NZ).astype(jnp.int32)
    valid = (ir >= 0) & (ir < NR) & (iz >= 0) & (iz < NZ)
    irc = jnp.clip(ir, 0, NR - 1)
    itc = jnp.clip(it, 0, NT - 1)
    izc = jnp.clip(iz, 0, NZ - 1)
    lin = (irc * NT + itc) * NZ + izc
    low = lin & 0xFFFF
    row = (low & 0xE000) | ((low & 0x1F) << 8) | ((low >> 5) & 0xFF)
    lin_ref[...] = jnp.where(valid, (lin - low) + row, BIG)


def _sc_body(lin_hbm, feats_hbm, out_hbm,
             seg_v, cvox, cpid, idx0, pid0, idx1, pid1, grows0, grows1,
             ones, zbuf, zocc, outbuf, obuf, rcp, sfeat, socc,
             gsem, lsem):
    cid = lax.axis_index("c")
    sid = lax.axis_index("s")
    pt_base = sid * NPT
    row0 = sid * SLICE
    iota = jnp.arange(16, dtype=jnp.int32)
    zf = jnp.zeros((16,), jnp.float32)
    chv = jnp.full((16,), CH, jnp.uint32)

    def fill_body(i, _):
        zbuf[i, :] = zf
        zocc[pl.ds(i * 16, 16)] = zf
        return 0
    lax.fori_loop(0, 64, fill_body, 0)
    for k in range(FLUSH // 16):
        ones[pl.ds(k * 16, 16)] = jnp.ones((16,), jnp.float32)

    def zero_slice():
        for j in range(SLICE // 64):
            pltpu.sync_copy(zbuf, sfeat.at[pl.ds(row0 + j * 64, 64)])
        for j in range(SLICE // 1024):
            pltpu.sync_copy(zocc, socc.at[pl.ds(row0 + j * 1024, 1024)])

    zero_slice()

    def build_block(b, idx_s, pid_s):
        o = (b * FLUSH) & (RING - 1)
        for k in range(FLUSH // 16):
            idx_s[pl.ds(k * 16, 16)] = cvox[pl.ds(o + k * 16, 16)]
            pid_s[pl.ds(k * 16, 16)] = cpid[pl.ds(o + k * 16, 16)]

    def fire_block(b):
        def go(idx_s, pid_s, grows):
            build_block(b, idx_s, pid_s)
            pltpu.async_copy(feats_hbm.at[pid_s], grows, gsem)

        @pl.when(b % 2 == 0)
        def _():
            go(idx0, pid0, grows0)

        @pl.when(b % 2 == 1)
        def _():
            go(idx1, pid1, grows1)

    def drain_block(b):
        def go(idx_s, pid_s, grows):
            pltpu.make_async_copy(feats_hbm.at[pid_s], grows, gsem).wait()
            pltpu.sync_copy(grows, sfeat.at[idx_s], add=True)
            pltpu.sync_copy(ones, socc.at[idx_s], add=True)

        @pl.when(b % 2 == 0)
        def _():
            go(idx0, pid0, grows0)

        @pl.when(b % 2 == 1)
        def _():
            go(idx1, pid1, grows1)

    def lin_slot(g):
        return g % 2

    def fire_lin(g):
        @pl.when(g < NSEG)
        def _():
            pltpu.async_copy(
                lin_hbm.at[pl.ds(pt_base + g * SEG, SEG)],
                seg_v.at[lin_slot(g)], lsem)

    def wait_lin(g):
        pltpu.make_async_copy(
            lin_hbm.at[pl.ds(pt_base + g * SEG, SEG)],
            seg_v.at[lin_slot(g)], lsem).wait()

    def round_body(rnd, _):
        plsc.subcore_barrier()
        base = (rnd * 2 + cid) * CH
        base_v = jnp.full((16,), base, jnp.int32)
        ringm = jnp.full((16,), RING - 1, jnp.int32)

        fire_lin(0)

        def seg_body(g, carry):
            cnt0, fired0, drained0 = carry
            wait_lin(g)
            fire_lin(g + 1)
            slot = lin_slot(g)
            seg_pt = pt_base + g * SEG

            def scan_body(i, cnt):
                l = seg_v[slot, pl.ds(i * 16, 16)]
                d = l - base_v
                m = d.astype(jnp.uint32) < chv
                inc = m.astype(jnp.int32)
                pos = (jnp.full((16,), cnt, jnp.int32)
                       + plsc.cumsum(inc) - inc) & ringm
                plsc.store_scatter(cvox, [pos], d, mask=m)
                plsc.store_scatter(cpid, [pos],
                                   iota + jnp.full((16,), seg_pt + i * 16,
                                                   jnp.int32), mask=m)
                return cnt + jnp.sum(inc)
            cnt = lax.fori_loop(0, SVR, scan_body, cnt0)

            def fl_body(j, carry):
                fired, drained = carry
                fire_block(fired)
                @pl.when(fired > drained)
                def _():
                    drain_block(drained)
                return (fired + 1,
                        jnp.where(fired > drained, drained + 1, drained))
            nfl = cnt // FLUSH - fired0
            fired, drained = lax.fori_loop(0, nfl, fl_body,
                                           (fired0, drained0))
            return (cnt, fired, drained)

        cnt, fired, drained = lax.fori_loop(
            0, NSEG, seg_body, (jnp.int32(0), jnp.int32(0), jnp.int32(0)))

        @pl.when(fired > drained)
        def _():
            drain_block(drained)

        @pl.when(cnt > fired * FLUSH)
        def _():
            trash_v = jnp.full((16,), TRASH, jnp.int32)
            zero_i = jnp.zeros((16,), jnp.int32)
            for k in range(FLUSH // 16):
                posk = (jnp.full((16,), cnt + k * 16, jnp.int32)
                        + iota) & ringm
                plsc.store_scatter(cvox, [posk], trash_v)
                plsc.store_scatter(cpid, [posk], zero_i)
            fire_block(fired)
            drain_block(fired)

        plsc.subcore_barrier()

        r_loc = sid // 2
        z0 = (sid % 2) * 16
        r = (rnd * 2 + cid) * 8 + r_loc
        vbuf = grows0.at[pl.ds(0, 256)]
        epsv = jnp.full((16,), 1e-8, jnp.float32)

        def fin_body(zi, _):
            z = z0 + zi
            rowz = (r_loc * 32 + z) * 256
            pltpu.sync_copy(sfeat.at[pl.ds(rowz, 256)], vbuf)
            pltpu.sync_copy(socc.at[pl.ds(rowz, 256)], obuf)

            def rcp_body(j, _):
                o = obuf[pl.ds(j * 16, 16)]
                rcp[pl.ds(j * 16, 16)] = jnp.where(
                    o > zf, jnp.ones((16,), jnp.float32) / (o + epsv), zf)
                return 0
            lax.fori_loop(0, 16, rcp_body, 0)

            def t_body(j, _):
                rv = rcp[pl.ds(j * 16, 16)]
                tl = jnp.full((16,), j * 16, jnp.int32) + iota
                for c in range(C):
                    g = plsc.load_gather(
                        vbuf, [tl, jnp.full((16,), c, jnp.int32)])
                    outbuf[c, pl.ds(j * 16, 16)] = g * rv
                return 0
            lax.fori_loop(0, 16, t_body, 0)
            pltpu.sync_copy(outbuf, out_hbm.at[r, z])
            return 0
        lax.fori_loop(0, 16, fin_body, 0)
        zero_slice()
        return 0

    lax.fori_loop(0, ROUNDS, round_body, 0)


_sc_voxelize = pl.kernel(
    _sc_body,
    out_type=jax.ShapeDtypeStruct((NR, NZ, C, NT), jnp.float32),
    mesh=plsc.VectorSubcoreMesh(core_axis_name="c", subcore_axis_name="s"),
    compiler_params=pltpu.CompilerParams(needs_layout_passes=False,
                                         use_tc_tiling_on_sc=False),
    scratch_types=[
        pltpu.VMEM((2, SEG), jnp.int32),
        pltpu.VMEM((RING,), jnp.int32),
        pltpu.VMEM((RING,), jnp.int32),
        pltpu.VMEM((FLUSH,), jnp.int32),
        pltpu.VMEM((FLUSH,), jnp.int32),
        pltpu.VMEM((FLUSH,), jnp.int32),
        pltpu.VMEM((FLUSH,), jnp.int32),
        pltpu.VMEM((FLUSH, C), jnp.float32),
        pltpu.VMEM((FLUSH, C), jnp.float32),
        pltpu.VMEM((FLUSH,), jnp.float32),
        pltpu.VMEM((64, C), jnp.float32),
        pltpu.VMEM((1024,), jnp.float32),
        pltpu.VMEM((C, NT), jnp.float32),
        pltpu.VMEM((256,), jnp.float32),
        pltpu.VMEM((256,), jnp.float32),
        pltpu.VMEM_SHARED((CH + 16, C), jnp.float32),
        pltpu.VMEM_SHARED((CH + 16,), jnp.float32),
        pltpu.SemaphoreType.DMA,
        pltpu.SemaphoreType.DMA,
    ],
)


def kernel(points, features):
    pad = PTS_PAD - points.shape[0]
    x = jnp.pad(points[:, 0], (0, pad), constant_values=1e9)
    y = jnp.pad(points[:, 1], (0, pad), constant_values=1e9)
    z = jnp.pad(points[:, 2], (0, pad), constant_values=1e9)

    BLK = 4096
    lin = pl.pallas_call(
        _binning_body,
        out_shape=jax.ShapeDtypeStruct((PTS_PAD,), jnp.int32),
        grid=(PTS_PAD // BLK,),
        in_specs=[pl.BlockSpec((BLK,), lambda i: (i,))] * 3,
        out_specs=pl.BlockSpec((BLK,), lambda i: (i,)),
    )(x, y, z)

    grid_zct = _sc_voxelize(lin, features)
    return grid_zct.transpose(0, 3, 1, 2)


# speedup vs baseline: 1.5346x; 1.0225x over previous
import jax
import jax.numpy as jnp
from jax import lax
from jax.experimental import pallas as pl
from jax.experimental.pallas import tpu as pltpu
from jax.experimental.pallas import tpu_sc as plsc

NR, NT, NZ = 128, 256, 32
R_MIN, R_MAX = 0.0, 50.0
Z_MIN, Z_MAX = -10.0, 10.0
C = 16
V = NR * NT * NZ

PTS_PAD = 409600
NPT = PTS_PAD // 16
SEG = 3200
NSEG = NPT // SEG
SVR = SEG // 16
NCH = 16
CH = V // NCH
ROUNDS = NCH // 2
SLICE = CH // 16
TRASH = CH
BIG = 1 << 30

RING = 8192
FLUSH = 512


def _binning_body(x_ref, y_ref, z_ref, lin_ref):
    x = x_ref[...]
    y = y_ref[...]
    z = z_ref[...]
    r = jnp.sqrt(x * x + y * y)
    theta = jnp.arctan2(y, x)
    theta = (theta + 2.0 * jnp.pi) % (2.0 * jnp.pi)
    ir = jnp.floor((r - R_MIN) / (R_MAX - R_MIN) * NR).astype(jnp.int32)
    it = jnp.floor(theta / (2.0 * jnp.pi) * NT).astype(jnp.int32) % NT
    iz = jnp.floor((z - Z_MIN) / (Z_MAX - Z_MIN) * NZ).astype(jnp.int32)
    valid = (ir >= 0) & (ir < NR) & (iz >= 0) & (iz < NZ)
    irc = jnp.clip(ir, 0, NR - 1)
    itc = jnp.clip(it, 0, NT - 1)
    izc = jnp.clip(iz, 0, NZ - 1)
    lin = (irc * NT + itc) * NZ + izc
    low = lin & 0xFFFF
    row = (low & 0xE000) | ((low & 0x1F) << 8) | ((low >> 5) & 0xFF)
    lin_ref[...] = jnp.where(valid, (lin - low) + row, BIG)


def _sc_body(lin_hbm, feats_hbm, out_hbm,
             seg_v, cvox, cpid, idx0, pid0, idx1, pid1, grows0, grows1,
             ones, zbuf, zocc, outbuf, obuf, rcp, sfeat, socc,
             gsem, lsem):
    cid = lax.axis_index("c")
    sid = lax.axis_index("s")
    pt_base = sid * NPT
    row0 = sid * SLICE
    iota = jnp.arange(16, dtype=jnp.int32)
    zf = jnp.zeros((16,), jnp.float32)
    chv = jnp.full((16,), CH, jnp.uint32)

    def fill_body(i, _):
        zbuf[i, :] = zf
        zocc[pl.ds(i * 16, 16)] = zf
        return 0
    lax.fori_loop(0, 256, fill_body, 0)
    for k in range(FLUSH // 16):
        ones[pl.ds(k * 16, 16)] = jnp.ones((16,), jnp.float32)

    def zero_slice():
        for j in range(SLICE // 256):
            pltpu.sync_copy(zbuf, sfeat.at[pl.ds(row0 + j * 256, 256)])
        for j in range(SLICE // 4096):
            pltpu.sync_copy(zocc, socc.at[pl.ds(row0 + j * 4096, 4096)])

    zero_slice()

    def build_block(b, idx_s, pid_s):
        o = (b * FLUSH) & (RING - 1)
        for k in range(FLUSH // 16):
            idx_s[pl.ds(k * 16, 16)] = cvox[pl.ds(o + k * 16, 16)]
            pid_s[pl.ds(k * 16, 16)] = cpid[pl.ds(o + k * 16, 16)]

    def fire_block(b):
        def go(idx_s, pid_s, grows):
            build_block(b, idx_s, pid_s)
            pltpu.async_copy(feats_hbm.at[pid_s], grows, gsem)

        @pl.when(b % 2 == 0)
        def _():
            go(idx0, pid0, grows0)

        @pl.when(b % 2 == 1)
        def _():
            go(idx1, pid1, grows1)

    def drain_block(b):
        def go(idx_s, pid_s, grows):
            pltpu.make_async_copy(feats_hbm.at[pid_s], grows, gsem).wait()
            pltpu.sync_copy(grows, sfeat.at[idx_s], add=True)
            pltpu.sync_copy(ones, socc.at[idx_s], add=True)

        @pl.when(b % 2 == 0)
        def _():
            go(idx0, pid0, grows0)

        @pl.when(b % 2 == 1)
        def _():
            go(idx1, pid1, grows1)

    def lin_slot(g):
        return g % 2

    def fire_lin(g):
        @pl.when(g < NSEG)
        def _():
            pltpu.async_copy(
                lin_hbm.at[pl.ds(pt_base + g * SEG, SEG)],
                seg_v.at[lin_slot(g)], lsem)

    def wait_lin(g):
        pltpu.make_async_copy(
            lin_hbm.at[pl.ds(pt_base + g * SEG, SEG)],
            seg_v.at[lin_slot(g)], lsem).wait()

    def round_body(rnd, _):
        plsc.subcore_barrier()
        base = (rnd * 2 + cid) * CH
        base_v = jnp.full((16,), base, jnp.int32)
        ringm = jnp.full((16,), RING - 1, jnp.int32)

        fire_lin(0)

        def seg_body(g, carry):
            cnt0, fired0, drained0 = carry
            wait_lin(g)
            fire_lin(g + 1)
            slot = lin_slot(g)
            seg_pt = pt_base + g * SEG

            def scan_body(i, cnt):
                l = seg_v[slot, pl.ds(i * 16, 16)]
                d = l - base_v
                m = d.astype(jnp.uint32) < chv
                inc = m.astype(jnp.int32)
                pos = (jnp.full((16,), cnt, jnp.int32)
                       + plsc.cumsum(inc) - inc) & ringm
                plsc.store_scatter(cvox, [pos], d, mask=m)
                plsc.store_scatter(cpid, [pos],
                                   iota + jnp.full((16,), seg_pt + i * 16,
                                                   jnp.int32), mask=m)
                return cnt + jnp.sum(inc)
            cnt = lax.fori_loop(0, SVR, scan_body, cnt0)

            def fl_body(j, carry):
                fired, drained = carry
                fire_block(fired)
                @pl.when(fired > drained)
                def _():
                    drain_block(drained)
                return (fired + 1,
                        jnp.where(fired > drained, drained + 1, drained))
            nfl = cnt // FLUSH - fired0
            fired, drained = lax.fori_loop(0, nfl, fl_body,
                                           (fired0, drained0))
            return (cnt, fired, drained)

        cnt, fired, drained = lax.fori_loop(
            0, NSEG, seg_body, (jnp.int32(0), jnp.int32(0), jnp.int32(0)))

        @pl.when(fired > drained)
        def _():
            drain_block(drained)

        @pl.when(cnt > fired * FLUSH)
        def _():
            trash_v = jnp.full((16,), TRASH, jnp.int32)
            zero_i = jnp.zeros((16,), jnp.int32)
            for k in range(FLUSH // 16):
                posk = (jnp.full((16,), cnt + k * 16, jnp.int32)
                        + iota) & ringm
                plsc.store_scatter(cvox, [posk], trash_v)
                plsc.store_scatter(cpid, [posk], zero_i)
            fire_block(fired)
            drain_block(fired)

        plsc.subcore_barrier()

        r_loc = sid // 2
        z0 = (sid % 2) * 16
        r = (rnd * 2 + cid) * 8 + r_loc
        vbuf = grows0.at[pl.ds(0, 256)]
        epsv = jnp.full((16,), 1e-8, jnp.float32)

        def fin_body(zi, _):
            z = z0 + zi
            rowz = (r_loc * 32 + z) * 256
            pltpu.sync_copy(sfeat.at[pl.ds(rowz, 256)], vbuf)
            pltpu.sync_copy(socc.at[pl.ds(rowz, 256)], obuf)

            def rcp_body(j, _):
                o = obuf[pl.ds(j * 16, 16)]
                rcp[pl.ds(j * 16, 16)] = jnp.where(
                    o > zf, jnp.ones((16,), jnp.float32) / (o + epsv), zf)
                return 0
            lax.fori_loop(0, 16, rcp_body, 0)

            def t_body(j, _):
                rv = rcp[pl.ds(j * 16, 16)]
                tl = jnp.full((16,), j * 16, jnp.int32) + iota
                for c in range(C):
                    g = plsc.load_gather(
                        vbuf, [tl, jnp.full((16,), c, jnp.int32)])
                    outbuf[c, pl.ds(j * 16, 16)] = g * rv
                return 0
            lax.fori_loop(0, 16, t_body, 0)
            pltpu.sync_copy(outbuf, out_hbm.at[r, z])
            return 0
        lax.fori_loop(0, 16, fin_body, 0)
        zero_slice()
        return 0

    lax.fori_loop(0, ROUNDS, round_body, 0)


_sc_voxelize = pl.kernel(
    _sc_body,
    out_type=jax.ShapeDtypeStruct((NR, NZ, C, NT), jnp.float32),
    mesh=plsc.VectorSubcoreMesh(core_axis_name="c", subcore_axis_name="s"),
    compiler_params=pltpu.CompilerParams(needs_layout_passes=False,
                                         use_tc_tiling_on_sc=False),
    scratch_types=[
        pltpu.VMEM((2, SEG), jnp.int32),
        pltpu.VMEM((RING,), jnp.int32),
        pltpu.VMEM((RING,), jnp.int32),
        pltpu.VMEM((FLUSH,), jnp.int32),
        pltpu.VMEM((FLUSH,), jnp.int32),
        pltpu.VMEM((FLUSH,), jnp.int32),
        pltpu.VMEM((FLUSH,), jnp.int32),
        pltpu.VMEM((FLUSH, C), jnp.float32),
        pltpu.VMEM((FLUSH, C), jnp.float32),
        pltpu.VMEM((FLUSH,), jnp.float32),
        pltpu.VMEM((256, C), jnp.float32),
        pltpu.VMEM((4096,), jnp.float32),
        pltpu.VMEM((C, NT), jnp.float32),
        pltpu.VMEM((256,), jnp.float32),
        pltpu.VMEM((256,), jnp.float32),
        pltpu.VMEM_SHARED((CH + 16, C), jnp.float32),
        pltpu.VMEM_SHARED((CH + 16,), jnp.float32),
        pltpu.SemaphoreType.DMA,
        pltpu.SemaphoreType.DMA,
    ],
)


def kernel(points, features):
    pad = PTS_PAD - points.shape[0]
    x = jnp.pad(points[:, 0], (0, pad), constant_values=1e9)
    y = jnp.pad(points[:, 1], (0, pad), constant_values=1e9)
    z = jnp.pad(points[:, 2], (0, pad), constant_values=1e9)

    BLK = 4096
    lin = pl.pallas_call(
        _binning_body,
        out_shape=jax.ShapeDtypeStruct((PTS_PAD,), jnp.int32),
        grid=(PTS_PAD // BLK,),
        in_specs=[pl.BlockSpec((BLK,), lambda i: (i,))] * 3,
        out_specs=pl.BlockSpec((BLK,), lambda i: (i,)),
    )(x, y, z)

    grid_zct = _sc_voxelize(lin, features)
    return grid_zct.transpose(0, 3, 1, 2)
